# baseline (speedup 1.0000x reference)
import functools

import jax
import jax.numpy as jnp
from jax import lax
from jax.experimental import pallas as pl
from jax.experimental.pallas import tpu as pltpu
from jax.experimental.pallas import tpu_sc as plsc

_IDX_BLK = 128
_PACK = 4


def kernel(user, item, user_emb, item_emb, user_bias, item_bias, global_bias):
    B = user.shape[0]
    D = user_emb.shape[1]
    info = plsc.get_sparse_core_info()
    nc, ns, L = info.num_cores, info.num_subcores, info.num_lanes
    nw = nc * ns
    bpw = B // nw
    nblk = bpw // _IDX_BLK
    cpb = _IDX_BLK // L

    ueg = user_emb.reshape(user_emb.shape[0] // _PACK, _PACK * D)
    ieg = item_emb.reshape(item_emb.shape[0] // _PACK, _PACK * D)
    u1 = user.astype(jnp.int32)
    i1 = item.astype(jnp.int32)
    ubf = user_bias.reshape(-1)
    ibf = item_bias.reshape(-1)
    gb16 = jnp.broadcast_to(global_bias.astype(jnp.float32), (L,))

    mesh = plsc.VectorSubcoreMesh(core_axis_name="c", subcore_axis_name="s")

    @functools.partial(
        pl.kernel,
        mesh=mesh,
        out_type=jax.ShapeDtypeStruct((B,), jnp.float32),
        compiler_params=pltpu.CompilerParams(needs_layout_passes=False),
        scratch_types=[
            pltpu.VMEM((bpw,), jnp.int32),
            pltpu.VMEM((bpw,), jnp.int32),
            pltpu.VMEM((nblk, _IDX_BLK), jnp.int32),
            pltpu.VMEM((nblk, _IDX_BLK), jnp.int32),
            pltpu.VMEM((_IDX_BLK, _PACK * D), jnp.float32),
            pltpu.VMEM((_IDX_BLK, _PACK * D), jnp.float32),
            pltpu.VMEM((bpw,), jnp.float32),
            pltpu.VMEM((bpw,), jnp.float32),
            pltpu.VMEM((bpw,), jnp.float32),
            pltpu.VMEM((L,), jnp.float32),
            pltpu.SemaphoreType.DMA,
            pltpu.SemaphoreType.DMA,
        ],
    )
    def mf(user_hbm, item_hbm, ue_hbm, ie_hbm, ub_hbm, ib_hbm, gb_hbm, out_hbm,
           uidx_v, iidx_v, urow_v, irow_v, ue_v, ie_v, ub_v, ib_v, out_v, gb_v,
           sem, sem2):
        wid = lax.axis_index("s") * nc + lax.axis_index("c")
        base = wid * bpw
        pltpu.sync_copy(user_hbm.at[pl.ds(base, bpw)], uidx_v)
        pltpu.sync_copy(item_hbm.at[pl.ds(base, bpw)], iidx_v)
        pltpu.sync_copy(gb_hbm, gb_v)

        for v in range(bpw // L):
            r0 = v * L
            uv = uidx_v[pl.ds(r0, L)]
            iv = iidx_v[pl.ds(r0, L)]
            urow_v[v // (_IDX_BLK // L), pl.ds(r0 % _IDX_BLK, L)] = (
                lax.shift_right_logical(uv, 2))
            irow_v[v // (_IDX_BLK // L), pl.ds(r0 % _IDX_BLK, L)] = (
                lax.shift_right_logical(iv, 2))

        bias_copies = []
        for j in range(nblk):
            sl = pl.ds(j * _IDX_BLK, _IDX_BLK)
            bias_copies.append(
                pltpu.async_copy(ub_hbm.at[uidx_v.at[sl]], ub_v.at[sl], sem2))
            bias_copies.append(
                pltpu.async_copy(ib_hbm.at[iidx_v.at[sl]], ib_v.at[sl], sem2))
        for cp in bias_copies:
            cp.wait()
        gvec = gb_v[...]

        for blk in range(nblk):
            cu = pltpu.async_copy(ue_hbm.at[urow_v.at[blk]], ue_v, sem)
            ci = pltpu.async_copy(ie_hbm.at[irow_v.at[blk]], ie_v, sem)
            cu.wait()
            ci.wait()

            def chunk(c, carry):
                r0 = blk * _IDX_BLK + c * L
                sl = pl.ds(r0, L)
                uv = uidx_v[sl]
                iv = iidx_v[sl]
                pvec = c * L + lax.iota(jnp.int32, L)
                ju = (uv & (_PACK - 1)) * D
                ji = (iv & (_PACK - 1)) * D
                accs = [ub_v[sl] + ib_v[sl] + gvec,
                        jnp.zeros((L,), jnp.float32),
                        jnp.zeros((L,), jnp.float32),
                        jnp.zeros((L,), jnp.float32)]
                for d in range(D):
                    u = plsc.load_gather(ue_v, [pvec, ju + d])
                    w = plsc.load_gather(ie_v, [pvec, ji + d])
                    accs[d % 4] = accs[d % 4] + u * w
                s = (accs[0] + accs[1]) + (accs[2] + accs[3])
                out_v[sl] = 1.0 / (1.0 + jnp.exp(-s))
                return carry

            lax.fori_loop(0, cpb, chunk, 0)

        pltpu.sync_copy(out_v, out_hbm.at[pl.ds(base, bpw)])

    return mf(u1, i1, ueg, ieg, ubf, ibf, gb16)


# speedup vs baseline: 1.0163x; 1.0163x over previous
import functools

import jax
import jax.numpy as jnp
from jax import lax
from jax.experimental import pallas as pl
from jax.experimental.pallas import tpu as pltpu
from jax.experimental.pallas import tpu_sc as plsc

_IDX_BLK = 128
_PACK = 4


def kernel(user, item, user_emb, item_emb, user_bias, item_bias, global_bias):
    B = user.shape[0]
    D = user_emb.shape[1]
    info = plsc.get_sparse_core_info()
    nc, ns, L = info.num_cores, info.num_subcores, info.num_lanes
    nw = nc * ns
    bpw = B // nw
    nblk = bpw // _IDX_BLK
    cpb = _IDX_BLK // L

    ueg = user_emb.reshape(user_emb.shape[0] // _PACK, _PACK * D)
    ieg = item_emb.reshape(item_emb.shape[0] // _PACK, _PACK * D)
    u1 = user.astype(jnp.int32)
    i1 = item.astype(jnp.int32)
    ubf = user_bias.reshape(-1)
    ibf = item_bias.reshape(-1)
    gb16 = jnp.broadcast_to(global_bias.astype(jnp.float32), (L,))

    mesh = plsc.VectorSubcoreMesh(core_axis_name="c", subcore_axis_name="s")

    @functools.partial(
        pl.kernel,
        mesh=mesh,
        out_type=jax.ShapeDtypeStruct((B,), jnp.float32),
        compiler_params=pltpu.CompilerParams(needs_layout_passes=False),
        scratch_types=[
            pltpu.VMEM((bpw,), jnp.int32),
            pltpu.VMEM((bpw,), jnp.int32),
            pltpu.VMEM((nblk, _IDX_BLK), jnp.int32),
            pltpu.VMEM((nblk, _IDX_BLK), jnp.int32),
            pltpu.VMEM((2, _IDX_BLK, _PACK * D), jnp.float32),
            pltpu.VMEM((2, _IDX_BLK, _PACK * D), jnp.float32),
            pltpu.VMEM((bpw,), jnp.float32),
            pltpu.VMEM((bpw,), jnp.float32),
            pltpu.VMEM((bpw,), jnp.float32),
            pltpu.VMEM((L,), jnp.float32),
            pltpu.SemaphoreType.DMA,
            pltpu.SemaphoreType.DMA,
            pltpu.SemaphoreType.DMA,
        ],
    )
    def mf(user_hbm, item_hbm, ue_hbm, ie_hbm, ub_hbm, ib_hbm, gb_hbm, out_hbm,
           uidx_v, iidx_v, urow_v, irow_v, ue_v, ie_v, ub_v, ib_v, out_v, gb_v,
           sem, sem2, sem3):
        wid = lax.axis_index("s") * nc + lax.axis_index("c")
        base = wid * bpw
        pltpu.sync_copy(user_hbm.at[pl.ds(base, bpw)], uidx_v)
        pltpu.sync_copy(item_hbm.at[pl.ds(base, bpw)], iidx_v)
        pltpu.sync_copy(gb_hbm, gb_v)

        for v in range(bpw // L):
            r0 = v * L
            uv = uidx_v[pl.ds(r0, L)]
            iv = iidx_v[pl.ds(r0, L)]
            urow_v[v // (_IDX_BLK // L), pl.ds(r0 % _IDX_BLK, L)] = (
                lax.shift_right_logical(uv, 2))
            irow_v[v // (_IDX_BLK // L), pl.ds(r0 % _IDX_BLK, L)] = (
                lax.shift_right_logical(iv, 2))

        bias_copies = []
        for j in range(nblk):
            sl = pl.ds(j * _IDX_BLK, _IDX_BLK)
            bias_copies.append(
                pltpu.async_copy(ub_hbm.at[uidx_v.at[sl]], ub_v.at[sl], sem2))
            bias_copies.append(
                pltpu.async_copy(ib_hbm.at[iidx_v.at[sl]], ib_v.at[sl], sem2))

        sems = [sem, sem3]
        pend = [None, None]
        pend[0] = (pltpu.async_copy(ue_hbm.at[urow_v.at[0]], ue_v.at[0], sem),
                   pltpu.async_copy(ie_hbm.at[irow_v.at[0]], ie_v.at[0], sem))

        for cp in bias_copies:
            cp.wait()
        gvec = gb_v[...]

        for blk in range(nblk):
            cur = blk % 2
            if blk + 1 < nblk:
                nxt = (blk + 1) % 2
                pend[nxt] = (
                    pltpu.async_copy(ue_hbm.at[urow_v.at[blk + 1]],
                                     ue_v.at[nxt], sems[nxt]),
                    pltpu.async_copy(ie_hbm.at[irow_v.at[blk + 1]],
                                     ie_v.at[nxt], sems[nxt]))
            pend[cur][0].wait()
            pend[cur][1].wait()

            def chunk(c, carry):
                r0 = blk * _IDX_BLK + c * L
                sl = pl.ds(r0, L)
                uv = uidx_v[sl]
                iv = iidx_v[sl]
                pvec = c * L + lax.iota(jnp.int32, L)
                ju = (uv & (_PACK - 1)) * D
                ji = (iv & (_PACK - 1)) * D
                accs = [ub_v[sl] + ib_v[sl] + gvec,
                        jnp.zeros((L,), jnp.float32),
                        jnp.zeros((L,), jnp.float32),
                        jnp.zeros((L,), jnp.float32)]
                for d in range(D):
                    u = plsc.load_gather(ue_v.at[cur], [pvec, ju + d])
                    w = plsc.load_gather(ie_v.at[cur], [pvec, ji + d])
                    accs[d % 4] = accs[d % 4] + u * w
                s = (accs[0] + accs[1]) + (accs[2] + accs[3])
                out_v[sl] = 1.0 / (1.0 + jnp.exp(-s))
                return carry

            lax.fori_loop(0, cpb, chunk, 0)

        pltpu.sync_copy(out_v, out_hbm.at[pl.ds(base, bpw)])

    return mf(u1, i1, ueg, ieg, ubf, ibf, gb16)
